# baseline (speedup 1.0000x reference)
import functools

import jax
import jax.numpy as jnp
from jax import lax
from jax.experimental import pallas as pl
from jax.experimental.pallas import tpu as pltpu
from jax.experimental.pallas import tpu_sc as plsc

B = 16384
D = 64
NC = 2
NS = 16
NW = NC * NS
BPW = B // NW
CHUNK = 128
NCHUNK = BPW // CHUNK
GROUPS = BPW // 16

_mesh = plsc.VectorSubcoreMesh(core_axis_name="c", subcore_axis_name="s")


@functools.partial(
    pl.kernel,
    mesh=_mesh,
    compiler_params=pltpu.CompilerParams(
        needs_layout_passes=False, use_tc_tiling_on_sc=False),
    out_type=jax.ShapeDtypeStruct((B,), jnp.float32),
    scratch_types=[
        pltpu.VMEM((BPW,), jnp.int32),
        pltpu.VMEM((BPW,), jnp.int32),
        pltpu.VMEM((BPW,), jnp.int32),
        pltpu.VMEM((BPW, D), jnp.float32),
        pltpu.VMEM((BPW, D), jnp.float32),
        pltpu.VMEM((BPW, D), jnp.float32),
        pltpu.VMEM((BPW,), jnp.float32),
        pltpu.VMEM((256,), jnp.float32),
        pltpu.SemaphoreType.DMA,
    ],
)
def _transe_kernel(heads_hbm, rel_hbm, tails_hbm, ent_hbm, relt_hbm,
                   out_hbm, h_idx, r_idx, t_idx, h_rows, r_rows, t_rows,
                   out_v, tbuf, sem):
    wid = lax.axis_index("s") * NC + lax.axis_index("c")
    base = wid * BPW

    pltpu.sync_copy(heads_hbm.at[pl.ds(base, BPW)], h_idx)
    pltpu.sync_copy(rel_hbm.at[pl.ds(base, BPW)], r_idx)
    pltpu.sync_copy(tails_hbm.at[pl.ds(base, BPW)], t_idx)

    copies = []
    for c in range(NCHUNK):
        sl = pl.ds(c * CHUNK, CHUNK)
        copies.append(pltpu.async_copy(ent_hbm.at[h_idx.at[sl]], h_rows.at[sl], sem))
        copies.append(pltpu.async_copy(relt_hbm.at[r_idx.at[sl]], r_rows.at[sl], sem))
        copies.append(pltpu.async_copy(ent_hbm.at[t_idx.at[sl]], t_rows.at[sl], sem))
    for cp in copies:
        cp.wait()

    lanes = lax.iota(jnp.int32, 16)

    colbase = lanes * 16

    def group_body(g, carry):
        for e in range(16):
            b = g * 16 + e
            acc = jnp.zeros((16,), jnp.float32)
            for c in range(D // 16):
                hv = h_rows[b, pl.ds(c * 16, 16)]
                rv = r_rows[b, pl.ds(c * 16, 16)]
                tv = t_rows[b, pl.ds(c * 16, 16)]
                d = (hv + rv) - tv
                acc = acc + d * d
            tbuf[pl.ds(e * 16, 16)] = acc
        tot = jnp.zeros((16,), jnp.float32)
        for k in range(16):
            tot = tot + plsc.load_gather(tbuf, [colbase + k])
        x = tot + 2e-38
        xi = plsc.bitcast(x, jnp.int32)
        y = plsc.bitcast((xi >> 1) + 0x1FBD1DF5, jnp.float32)
        y = 0.5 * (y + x / y)
        y = 0.5 * (y + x / y)
        out_v[pl.ds(pl.multiple_of(g * 16, 16), 16)] = -y
        return carry

    lax.fori_loop(0, GROUPS, group_body, 0)
    pltpu.sync_copy(out_v, out_hbm.at[pl.ds(base, BPW)])


def kernel(heads, relations, tails, entity_table, relation_table):
    return _transe_kernel(heads, relations, tails, entity_table,
                          relation_table)
